# Initial kernel scaffold; baseline (speedup 1.0000x reference)
import jax
import jax.numpy as jnp
from jax.experimental import pallas as pl


def kernel(inputs, targets, alpha):
    raise NotImplementedError("write your pallas kernel here")


# speedup vs baseline: 1.6353x; 1.6353x over previous
import functools

import jax
import jax.numpy as jnp
from jax import lax
from jax.experimental import pallas as pl
from jax.experimental.pallas import tpu as pltpu
from jax.experimental.pallas import tpu_sc as plsc

_GAMMA = 2.0

_NC = 2
_NS = 16
_LANES = 16
_NW = _NC * _NS


def _sc_gather(x_flat, tgt, alpha_flat, n_rows, n_cols):
    bpw = n_rows // _NW

    mesh = plsc.VectorSubcoreMesh(core_axis_name="c", subcore_axis_name="s")

    @functools.partial(
        pl.kernel,
        mesh=mesh,
        out_type=[
            jax.ShapeDtypeStruct((n_rows,), jnp.float32),
            jax.ShapeDtypeStruct((n_rows,), jnp.float32),
        ],
        scratch_types=[
            pltpu.VMEM((bpw,), jnp.int32),
            pltpu.VMEM((bpw,), jnp.int32),
            pltpu.VMEM((bpw,), jnp.float32),
            pltpu.VMEM((bpw,), jnp.float32),
            pltpu.SemaphoreType.DMA,
            pltpu.SemaphoreType.DMA,
        ],
    )
    def gather_kernel(x_hbm, tgt_hbm, alpha_hbm, xt_hbm, at_hbm,
                      tgt_v, idx_v, xt_v, at_v, sem_x, sem_a):
        wid = lax.axis_index("s") * _NC + lax.axis_index("c")
        base = wid * bpw
        pltpu.sync_copy(tgt_hbm.at[pl.ds(base, bpw)], tgt_v)

        @pl.loop(0, bpw, step=_LANES)
        def _(c):
            lane = lax.iota(jnp.int32, _LANES)
            rows = base + c + lane
            idx_v[pl.ds(c, _LANES)] = tgt_v[pl.ds(c, _LANES)] + rows * n_cols

        cp_x = pltpu.async_copy(x_hbm.at[idx_v], xt_v, sem_x)
        cp_a = pltpu.async_copy(alpha_hbm.at[tgt_v], at_v, sem_a)
        cp_x.wait()
        cp_a.wait()
        pltpu.sync_copy(xt_v, xt_hbm.at[pl.ds(base, bpw)])
        pltpu.sync_copy(at_v, at_hbm.at[pl.ds(base, bpw)])

    return gather_kernel(x_flat, tgt, alpha_flat)


def _tc_loss(inputs, xt, at, block_rows):
    n_rows, n_cols = inputs.shape
    grid = n_rows // block_rows
    inv_n = 1.0 / n_rows

    def body(x_ref, xt_ref, at_ref, out_ref):
        i = pl.program_id(0)
        x = x_ref[...]
        m = jnp.max(x, axis=1, keepdims=True)
        s = jnp.sum(jnp.exp(x - m), axis=1, keepdims=True)
        logp = xt_ref[...] - m - jnp.log(s)
        p = jnp.exp(logp)
        q = 1.0 - p
        partial = jnp.sum(at_ref[...] * q * q * logp)

        @pl.when(i == 0)
        def _():
            out_ref[0, 0] = 0.0

        out_ref[0, 0] += -inv_n * partial

    return pl.pallas_call(
        body,
        grid=(grid,),
        in_specs=[
            pl.BlockSpec((block_rows, n_cols), lambda i: (i, 0)),
            pl.BlockSpec((block_rows, 1), lambda i: (i, 0)),
            pl.BlockSpec((block_rows, 1), lambda i: (i, 0)),
        ],
        out_specs=pl.BlockSpec((1, 1), lambda i: (0, 0),
                               memory_space=pltpu.SMEM),
        out_shape=jax.ShapeDtypeStruct((1, 1), jnp.float32),
    )(inputs, xt, at)


def kernel(inputs, targets, alpha):
    n_rows, n_cols = inputs.shape
    tgt = targets.astype(jnp.int32)
    xt, at = _sc_gather(inputs.reshape(-1), tgt, alpha.reshape(-1),
                        n_rows, n_cols)
    loss = _tc_loss(inputs, xt.reshape(n_rows, 1), at.reshape(n_rows, 1),
                    block_rows=512)
    return loss[0, 0]


# speedup vs baseline: 3.1869x; 1.9488x over previous
import jax
import jax.numpy as jnp
from jax.experimental import pallas as pl
from jax.experimental.pallas import tpu as pltpu


def _tc_loss(inputs, tgt, alpha_row, block_rows):
    n_rows, n_cols = inputs.shape
    grid = n_rows // block_rows
    inv_n = 1.0 / n_rows

    def body(x_ref, tgt_ref, a_ref, out_ref):
        i = pl.program_id(0)
        x = x_ref[...]
        col = jax.lax.broadcasted_iota(jnp.int32, (block_rows, n_cols), 1)
        maskf = (col == tgt_ref[...]).astype(jnp.float32)
        xt = jnp.sum(x * maskf, axis=1, keepdims=True)
        at = jnp.sum(a_ref[...] * maskf, axis=1, keepdims=True)
        m = jnp.max(x, axis=1, keepdims=True)
        s = jnp.sum(jnp.exp(x - m), axis=1, keepdims=True)
        logp = xt - m - jnp.log(s)
        p = jnp.exp(logp)
        q = 1.0 - p
        partial = jnp.sum(at * q * q * logp)

        @pl.when(i == 0)
        def _():
            out_ref[0, 0] = 0.0

        out_ref[0, 0] += -inv_n * partial

    return pl.pallas_call(
        body,
        grid=(grid,),
        in_specs=[
            pl.BlockSpec((block_rows, n_cols), lambda i: (i, 0)),
            pl.BlockSpec((block_rows, 1), lambda i: (i, 0)),
            pl.BlockSpec((1, n_cols), lambda i: (0, 0)),
        ],
        out_specs=pl.BlockSpec((1, 1), lambda i: (0, 0),
                               memory_space=pltpu.SMEM),
        out_shape=jax.ShapeDtypeStruct((1, 1), jnp.float32),
    )(inputs, tgt, alpha_row)


def kernel(inputs, targets, alpha):
    n_rows, n_cols = inputs.shape
    tgt = targets.astype(jnp.int32).reshape(n_rows, 1)
    loss = _tc_loss(inputs, tgt, alpha.reshape(1, n_cols), block_rows=512)
    return loss[0, 0]
